# Initial kernel scaffold; baseline (speedup 1.0000x reference)
import jax
import jax.numpy as jnp
from jax.experimental import pallas as pl


def kernel(node_features, netlist_metadata, sparse_adj_weight, params, sparse_adj_i, sparse_adj_j, current_node):
    raise NotImplementedError("write your pallas kernel here")


# speedup vs baseline: 9.4226x; 9.4226x over previous
import functools

import jax
import jax.numpy as jnp
import numpy as np
from jax import lax
from jax.experimental import pallas as pl
from jax.experimental.pallas import tpu as pltpu
from jax.experimental.pallas import tpu_sc as plsc

_N = 10000
_E = 320000
_DF = 128
_H = 8
_GRID = 128
_EPS = 1e-6
_f32 = jnp.float32
_HI = lax.Precision.HIGHEST

_NC = 2
_NS = 16
_NW = _NC * _NS
_K = 80
_EPW = _E // _NW
_NCH = _EPW // _K
_RPT = _N // _NS

_GDN = lax.GatherDimensionNumbers(
    offset_dims=(), collapsed_slice_dims=(0,), start_index_map=(0,))


def _g1d(x, idx):
    return lax.gather(x, idx[:, None], _GDN, slice_sizes=(1,),
                      mode=lax.GatherScatterMode.PROMISE_IN_BOUNDS)


def _sc_layer(pq, qp, ii, jj, wv, cb, zeros_n16):
    mesh = plsc.VectorSubcoreMesh(core_axis_name="c", subcore_axis_name="s")

    @functools.partial(
        pl.kernel,
        out_type=jax.ShapeDtypeStruct((2 * _N, 16), _f32),
        mesh=mesh,
        compiler_params=pltpu.CompilerParams(use_tc_tiling_on_sc=False),
        scratch_types=[
            pltpu.VMEM((_K,), jnp.int32),
            pltpu.VMEM((_K,), jnp.int32),
            pltpu.VMEM((_K,), _f32),
            pltpu.VMEM((_K, 16), _f32),
            pltpu.VMEM((_K, 16), _f32),
            pltpu.VMEM((_K, 16), _f32),
            pltpu.VMEM((2, 16), _f32),
            pltpu.VMEM_SHARED((_N, 16), _f32),
            pltpu.SemaphoreType.DMA,
            pltpu.SemaphoreType.DMA,
        ],
    )
    def k(pq_h, qp_h, ii_h, jj_h, w_h, cb_h, z_h, out_h,
          i1, i2, wvv, b1, b2, hb, cbv, acc, s1, s2):
        cid = lax.axis_index("c")
        sid = lax.axis_index("s")
        wid = cid * _NS + sid
        pltpu.sync_copy(cb_h, cbv)
        pltpu.sync_copy(z_h.at[pl.ds(sid * _RPT, _RPT)],
                        acc.at[pl.ds(sid * _RPT, _RPT)])
        plsc.subcore_barrier()
        base = wid * _EPW
        swap = lax.iota(jnp.int32, 16) ^ 8
        hi = lax.iota(jnp.int32, 16) >= 8

        def body(t, carry):
            off = base + t * _K
            pltpu.sync_copy(ii_h.at[pl.ds(off, _K)], i1)
            pltpu.sync_copy(jj_h.at[pl.ds(off, _K)], i2)
            pltpu.sync_copy(w_h.at[pl.ds(off, _K)], wvv)
            c1 = pltpu.async_copy(pq_h.at[i1], b1, s1)
            c2 = pltpu.async_copy(qp_h.at[i2], b2, s2)
            c1.wait()
            c2.wait()
            cb0 = cbv[0, :]
            cb1 = cbv[1, :]
            for g in range(_K // 16):
                w16 = wvv[pl.ds(g * 16, 16)]
                for r16 in range(16):
                    r = g * 16 + r16
                    s = w16[r16]
                    mf = jnp.where(s != 0.0, 1.0, 0.0)
                    x = b1[r, :] + b2[r, :]
                    z = x * mf + (cb0 * s + cb1)
                    f = jnp.maximum(z, 0.0)
                    h2 = (f + _g1d(f, swap)) * 0.5
                    hb[r, :] = jnp.where(hi, 1.0, h2)
            pltpu.sync_copy(hb, acc.at[i1], add=True)
            pltpu.sync_copy(hb, acc.at[i2], add=True)
            return carry

        lax.fori_loop(0, _NCH, body, 0)
        plsc.subcore_barrier()
        pltpu.sync_copy(acc.at[pl.ds(sid * _RPT, _RPT)],
                        out_h.at[pl.ds(cid * _N + sid * _RPT, _RPT)])

    return k(pq, qp, ii, jj, wv, cb, zeros_n16)


def _k0_body(nf_ref, wf_ref, bf_ref, ab_ref, ba_ref, pq_ref, qp_ref):
    h = jnp.maximum(
        jnp.dot(nf_ref[...], wf_ref[...], precision=_HI) + bf_ref[...], 0.0)
    pq_ref[...] = jnp.dot(h, ab_ref[...], precision=_HI)
    qp_ref[...] = jnp.dot(h, ba_ref[...], precision=_HI)


def _tc_k0(nf, wf, bf, ab, ba):
    blk = 2000
    return pl.pallas_call(
        _k0_body,
        grid=(_N // blk,),
        in_specs=[
            pl.BlockSpec((blk, _DF), lambda i: (i, 0)),
            pl.BlockSpec((_DF, _H), lambda i: (0, 0)),
            pl.BlockSpec((1, _H), lambda i: (0, 0)),
            pl.BlockSpec((_H, 16), lambda i: (0, 0)),
            pl.BlockSpec((_H, 16), lambda i: (0, 0)),
        ],
        out_specs=[
            pl.BlockSpec((blk, 16), lambda i: (i, 0)),
            pl.BlockSpec((blk, 16), lambda i: (i, 0)),
        ],
        out_shape=[jax.ShapeDtypeStruct((_N, 16), _f32)] * 2,
    )(nf, wf, bf, ab, ba)


def _prep_body(parts_ref, p_ref, wab_ref, wba_ref, pq_ref, qp_ref):
    half = _N * 16 // 128
    tsum = parts_ref[0:half, :] + parts_ref[half:2 * half, :]
    cnt = jnp.dot(tsum, p_ref[...], precision=_HI)
    h8 = tsum / (cnt + _EPS)
    pq_ref[...] = jnp.dot(h8, wab_ref[...], precision=_HI)
    qp_ref[...] = jnp.dot(h8, wba_ref[...], precision=_HI)


def _tc_prep(parts8, pmat, wab, wba):
    half = _N * 16 // 128
    return pl.pallas_call(
        _prep_body,
        out_shape=[jax.ShapeDtypeStruct((half, 128), _f32)] * 2,
    )(parts8, pmat, wab, wba)


def _bf(x):
    return x.astype(jnp.bfloat16).astype(_f32)


def _tail_body(parts_ref, meta_ref, cur_ref, wm, bm, wq, bq, wk, bk, wv, bv,
               w1, b1, w2, b2, w3, b3, wp, bp, pol_ref, val_ref):
    t = parts_ref[0] + parts_ref[1]
    h3 = t[:, :_H] / (t[:, _H:] + _EPS)
    ids = lax.broadcasted_iota(jnp.int32, (1, _N), 1)
    onehot = (ids == cur_ref[...]).astype(_f32)
    h_cur = jnp.dot(onehot, h3, precision=_HI)
    h_meta = jnp.maximum(
        jnp.dot(_bf(meta_ref[...]), _bf(wm[...]), precision=_HI)
        + bm[...], 0.0)
    q = jnp.dot(_bf(h_cur), _bf(wq[...]), precision=_HI) + bq[...]
    kk = jnp.dot(h3, wk[...], precision=_HI) + bk[...]
    vv = jnp.dot(h3, wv[...], precision=_HI) + bv[...]
    s = jnp.sum(_bf(kk) * _bf(q), axis=1, keepdims=True)
    m = jnp.max(s, axis=0, keepdims=True)
    e = jnp.exp(s - m)
    att = e / jnp.sum(e, axis=0, keepdims=True)
    h_att = jnp.sum(_bf(vv) * _bf(att), axis=0, keepdims=True)
    h = jnp.concatenate([h_meta, h_cur, h_att], axis=1)
    u = jnp.maximum(
        jnp.dot(_bf(h), _bf(w1[...]), precision=_HI) + b1[...], 0.0)
    u = jnp.maximum(
        jnp.dot(_bf(u), _bf(w2[...]), precision=_HI) + b2[...], 0.0)
    val_ref[...] = jnp.dot(_bf(u), _bf(w3[...]), precision=_HI) + b3[...]
    pol_ref[...] = jnp.maximum(
        jnp.dot(_bf(h), _bf(wp[...]), precision=_HI) + bp[...], 0.0)


def _tc_tail(parts, meta, cur, wm, bm, wq, bq, wk, bk, wv, bv,
             w1, b1, w2, b2, w3, b3, wp, bp):
    npo = (_GRID // 16) * (_GRID // 16) * 32
    return pl.pallas_call(
        _tail_body,
        out_shape=[jax.ShapeDtypeStruct((1, npo), _f32),
                   jax.ShapeDtypeStruct((1, 1), _f32)],
    )(parts, meta, cur, wm, bm, wq, bq, wk, bk, wv, bv,
      w1, b1, w2, b2, w3, b3, wp, bp)


def _dc_body4(x_ref, w_ref, b_ref, o_ref):
    x = x_ref[...]
    for p in range(4):
        o_ref[p] = jnp.maximum(
            jnp.dot(x, w_ref[p], precision=_HI) + b_ref[...], 0.0)


def _tc_deconv(xcat, wstack, bias, hw, co):
    return pl.pallas_call(
        _dc_body4,
        out_shape=jax.ShapeDtypeStruct((4, hw, co), _f32),
    )(xcat, wstack, bias)


def _conv_body(pl_ref, w_ref, b_ref, o_ref):
    acc = pl_ref[0] * w_ref[0:1, :]
    for t in range(1, 18):
        acc = acc + pl_ref[t] * w_ref[t:t + 1, :]
    o_ref[...] = acc + b_ref[...]


def _tc_conv_final(planes, wvec, bias):
    return pl.pallas_call(
        _conv_body,
        out_shape=jax.ShapeDtypeStruct((_GRID, _GRID), _f32),
    )(planes, wvec, bias)


def _shifts2(x):
    h, w, ci = x.shape
    xp = jnp.pad(x, ((1, 0), (1, 0), (0, 0)))
    mats = [xp[:h, :w], xp[:h, 1:], xp[1:, :w], x]
    return jnp.concatenate([m.reshape(h * w, ci) for m in mats], axis=1)


def _wstack2(kt):
    z = jnp.zeros_like(kt[0, 0])
    w00 = jnp.concatenate([kt[0, 0], kt[0, 2], kt[2, 0], kt[2, 2]], axis=0)
    w01 = jnp.concatenate([z, kt[0, 1], z, kt[2, 1]], axis=0)
    w10 = jnp.concatenate([z, z, kt[1, 0], kt[1, 2]], axis=0)
    w11 = jnp.concatenate([z, z, z, kt[1, 1]], axis=0)
    return jnp.stack([w00, w01, w10, w11])


def _interleave(y4, h, w, co):
    y = y4.reshape(2, 2, h, w, co)
    return y.transpose(2, 0, 3, 1, 4).reshape(2 * h, 2 * w, co)


def _shift_planes(x):
    h, w, ci = x.shape
    xp = jnp.pad(x, ((1, 1), (1, 1), (0, 0)))
    return jnp.stack([xp[di:di + h, dj:dj + w, c]
                      for di in range(3) for dj in range(3)
                      for c in range(ci)])


_P16 = np.zeros((16, 16), np.float32)
for _k_ in range(16):
    _P16[8 + (_k_ % 8), _k_] = 1.0
_PMAT = np.kron(np.eye(8, dtype=np.float32), _P16)


def _blockdiag(w16):
    return jnp.kron(jnp.eye(8, dtype=_f32), w16)


def kernel(node_features, netlist_metadata, sparse_adj_weight, params,
           sparse_adj_i, sparse_adj_j, current_node):
    p = params
    nf = node_features[0]
    wvec = sparse_adj_weight[0].reshape(_E)
    ii = sparse_adj_i[0].astype(jnp.int32)
    jj = sparse_adj_j[0].astype(jnp.int32)
    cur = current_node.astype(jnp.int32).reshape(1, 1)

    abs_, bas_, cbs_ = [], [], []
    for l in range(3):
        we = p['We'][l]
        be = p['be'][l]
        a_, b_, c_ = we[:_H], we[_H:2 * _H], we[2 * _H]
        abs_.append(jnp.concatenate([a_, b_], axis=1))
        bas_.append(jnp.concatenate([b_, a_], axis=1))
        cbs_.append(jnp.stack([jnp.concatenate([c_, c_]),
                               jnp.concatenate([be, be])]))
    zeros_n16 = jnp.zeros((_N, 16), _f32)

    pq, qp = _tc_k0(nf, p['Wf'], p['bf'].reshape(1, _H), abs_[0], bas_[0])
    parts = None
    for l in range(3):
        parts = _sc_layer(pq, qp, ii, jj, wvec, cbs_[l], zeros_n16)
        if l < 2:
            parts8 = parts.reshape(2 * _N * 16 // 128, 128)
            wab = _blockdiag(jnp.concatenate(
                [abs_[l + 1], jnp.zeros((_H, 16), _f32)], axis=0))
            wba = _blockdiag(jnp.concatenate(
                [bas_[l + 1], jnp.zeros((_H, 16), _f32)], axis=0))
            pq8, qp8 = _tc_prep(parts8, jnp.asarray(_PMAT), wab, wba)
            pq = pq8.reshape(_N, 16)
            qp = qp8.reshape(_N, 16)

    pol, val = _tc_tail(
        parts.reshape(2, _N, 16), netlist_metadata, cur,
        p['Wm'], p['bm'].reshape(1, -1),
        p['Wq'], p['bq'].reshape(1, -1),
        p['Wk'], p['bk'].reshape(1, -1),
        p['Wv'], p['bv'].reshape(1, -1),
        p['Wv1'], p['bv1'].reshape(1, -1),
        p['Wv2'], p['bv2'].reshape(1, -1),
        p['Wv3'], p['bv3'].reshape(1, -1),
        p['Wp'], p['bp'].reshape(1, -1))

    x = pol.reshape(_GRID // 16, _GRID // 16, 32)
    for kt, cb in ((p['K1'], p['c1']), (p['K2'], p['c2']),
                   (p['K3'], p['c3']), (p['K4'], p['c4'])):
        h, w, ci = x.shape
        co = kt.shape[3]
        y4 = _tc_deconv(_shifts2(x), _wstack2(kt), cb.reshape(1, -1),
                        hw=h * w, co=co)
        x = _interleave(y4, h, w, co)

    y = _tc_conv_final(_shift_planes(x), p['K5'].reshape(18, 1),
                       p['c5'].reshape(1, 1))
    logits = y.reshape(1, _GRID * _GRID)
    return logits, val


# speedup vs baseline: 12.6619x; 1.3438x over previous
import functools

import jax
import jax.numpy as jnp
import numpy as np
from jax import lax
from jax.experimental import pallas as pl
from jax.experimental.pallas import tpu as pltpu
from jax.experimental.pallas import tpu_sc as plsc

_N = 10000
_E = 320000
_DF = 128
_H = 8
_GRID = 128
_EPS = 1e-6
_f32 = jnp.float32
_HI = lax.Precision.HIGHEST

_NC = 2
_NS = 16
_NW = _NC * _NS
_K = 80
_EPW = _E // _NW
_NCH = _EPW // _K
_RPT = _N // _NS

_GDN = lax.GatherDimensionNumbers(
    offset_dims=(), collapsed_slice_dims=(0,), start_index_map=(0,))


def _g1d(x, idx):
    return lax.gather(x, idx[:, None], _GDN, slice_sizes=(1,),
                      mode=lax.GatherScatterMode.PROMISE_IN_BOUNDS)


def _sc_layer(pq, qp, ii, jj, wv, cb, zeros_n16):
    mesh = plsc.VectorSubcoreMesh(core_axis_name="c", subcore_axis_name="s")

    @functools.partial(
        pl.kernel,
        out_type=jax.ShapeDtypeStruct((2 * _N, 16), _f32),
        mesh=mesh,
        compiler_params=pltpu.CompilerParams(use_tc_tiling_on_sc=False),
        scratch_types=[
            pltpu.VMEM((_K,), jnp.int32),
            pltpu.VMEM((_K,), jnp.int32),
            pltpu.VMEM((_K,), _f32),
            pltpu.VMEM((_K, 16), _f32),
            pltpu.VMEM((_K, 16), _f32),
            pltpu.VMEM((_K, 16), _f32),
            pltpu.VMEM((2, 16), _f32),
            pltpu.VMEM_SHARED((_N, 16), _f32),
            pltpu.SemaphoreType.DMA,
            pltpu.SemaphoreType.DMA,
            pltpu.SemaphoreType.DMA,
        ],
    )
    def k(pq_h, qp_h, ii_h, jj_h, w_h, cb_h, z_h, out_h,
          i1, i2, wvv, b1, b2, hb, cbv, acc, s1, s2, s3):
        cid = lax.axis_index("c")
        sid = lax.axis_index("s")
        wid = cid * _NS + sid
        pltpu.sync_copy(cb_h, cbv)
        pltpu.sync_copy(z_h.at[pl.ds(sid * _RPT, _RPT)],
                        acc.at[pl.ds(sid * _RPT, _RPT)])
        plsc.subcore_barrier()
        base = wid * _EPW
        swap = lax.iota(jnp.int32, 16) ^ 8
        hi = lax.iota(jnp.int32, 16) >= 8
        cb0 = cbv[0, :]
        cb1 = cbv[1, :]

        def process(t):
            off = base + t * _K
            l1 = pltpu.async_copy(ii_h.at[pl.ds(off, _K)], i1, s1)
            l2 = pltpu.async_copy(jj_h.at[pl.ds(off, _K)], i2, s1)
            l3 = pltpu.async_copy(w_h.at[pl.ds(off, _K)], wvv, s1)
            l1.wait()
            l2.wait()
            l3.wait()
            g1 = pltpu.async_copy(pq_h.at[i1], b1, s2)
            g2 = pltpu.async_copy(qp_h.at[i2], b2, s2)
            g1.wait()
            g2.wait()
            for g in range(_K // 16):
                w16 = wvv[pl.ds(g * 16, 16)]
                for r16 in range(16):
                    r = g * 16 + r16
                    s = w16[r16]
                    mf = jnp.where(s != 0.0, 1.0, 0.0)
                    x = b1[r, :] + b2[r, :]
                    z = x * mf + (cb0 * s + cb1)
                    f = jnp.maximum(z, 0.0)
                    h2 = (f + _g1d(f, swap)) * 0.5
                    hb[r, :] = jnp.where(hi, 1.0, h2)
            pltpu.async_copy(hb, acc.at[i1], s3, add=True)
            pltpu.async_copy(hb, acc.at[i2], s3, add=True)

        def drain_scatter():
            pltpu.make_async_copy(hb, acc.at[i1], s3).wait()
            pltpu.make_async_copy(hb, acc.at[i2], s3).wait()

        process(0)

        def body(t, carry):
            drain_scatter()
            process(t)
            return carry

        lax.fori_loop(1, _NCH, body, 0)
        drain_scatter()
        plsc.subcore_barrier()
        pltpu.sync_copy(acc.at[pl.ds(sid * _RPT, _RPT)],
                        out_h.at[pl.ds(cid * _N + sid * _RPT, _RPT)])

    return k(pq, qp, ii, jj, wv, cb, zeros_n16)


def _k0_body(nf_ref, wf_ref, bf_ref, ab_ref, ba_ref, pq_ref, qp_ref):
    h = jnp.maximum(
        jnp.dot(nf_ref[...], wf_ref[...], precision=_HI) + bf_ref[...], 0.0)
    pq_ref[...] = jnp.dot(h, ab_ref[...], precision=_HI)
    qp_ref[...] = jnp.dot(h, ba_ref[...], precision=_HI)


def _tc_k0(nf, wf, bf, ab, ba):
    blk = 2000
    return pl.pallas_call(
        _k0_body,
        grid=(_N // blk,),
        in_specs=[
            pl.BlockSpec((blk, _DF), lambda i: (i, 0)),
            pl.BlockSpec((_DF, _H), lambda i: (0, 0)),
            pl.BlockSpec((1, _H), lambda i: (0, 0)),
            pl.BlockSpec((_H, 16), lambda i: (0, 0)),
            pl.BlockSpec((_H, 16), lambda i: (0, 0)),
        ],
        out_specs=[
            pl.BlockSpec((blk, 16), lambda i: (i, 0)),
            pl.BlockSpec((blk, 16), lambda i: (i, 0)),
        ],
        out_shape=[jax.ShapeDtypeStruct((_N, 16), _f32)] * 2,
    )(nf, wf, bf, ab, ba)


def _prep_body(parts_ref, p_ref, wab_ref, wba_ref, pq_ref, qp_ref):
    half = _N * 16 // 128
    tsum = parts_ref[0:half, :] + parts_ref[half:2 * half, :]
    cnt = jnp.dot(tsum, p_ref[...], precision=_HI)
    h8 = tsum / (cnt + _EPS)
    pq_ref[...] = jnp.dot(h8, wab_ref[...], precision=_HI)
    qp_ref[...] = jnp.dot(h8, wba_ref[...], precision=_HI)


def _tc_prep(parts8, pmat, wab, wba):
    half = _N * 16 // 128
    return pl.pallas_call(
        _prep_body,
        out_shape=[jax.ShapeDtypeStruct((half, 128), _f32)] * 2,
    )(parts8, pmat, wab, wba)


def _bf(x):
    return x.astype(jnp.bfloat16).astype(_f32)


def _tail_body(parts_ref, meta_ref, cur_ref, wm, bm, wq, bq, wk, bk, wv, bv,
               w1, b1, w2, b2, w3, b3, wp, bp, pol_ref, val_ref):
    t = parts_ref[0] + parts_ref[1]
    h3 = t[:, :_H] / (t[:, _H:] + _EPS)
    ids = lax.broadcasted_iota(jnp.int32, (1, _N), 1)
    onehot = (ids == cur_ref[...]).astype(_f32)
    h_cur = jnp.dot(onehot, h3, precision=_HI)
    h_meta = jnp.maximum(
        jnp.dot(_bf(meta_ref[...]), _bf(wm[...]), precision=_HI)
        + bm[...], 0.0)
    q = jnp.dot(_bf(h_cur), _bf(wq[...]), precision=_HI) + bq[...]
    kk = jnp.dot(h3, wk[...], precision=_HI) + bk[...]
    vv = jnp.dot(h3, wv[...], precision=_HI) + bv[...]
    s = jnp.sum(_bf(kk) * _bf(q), axis=1, keepdims=True)
    m = jnp.max(s, axis=0, keepdims=True)
    e = jnp.exp(s - m)
    att = e / jnp.sum(e, axis=0, keepdims=True)
    h_att = jnp.sum(_bf(vv) * _bf(att), axis=0, keepdims=True)
    h = jnp.concatenate([h_meta, h_cur, h_att], axis=1)
    u = jnp.maximum(
        jnp.dot(_bf(h), _bf(w1[...]), precision=_HI) + b1[...], 0.0)
    u = jnp.maximum(
        jnp.dot(_bf(u), _bf(w2[...]), precision=_HI) + b2[...], 0.0)
    val_ref[...] = jnp.dot(_bf(u), _bf(w3[...]), precision=_HI) + b3[...]
    pol_ref[...] = jnp.maximum(
        jnp.dot(_bf(h), _bf(wp[...]), precision=_HI) + bp[...], 0.0)


def _tc_tail(parts, meta, cur, wm, bm, wq, bq, wk, bk, wv, bv,
             w1, b1, w2, b2, w3, b3, wp, bp):
    npo = (_GRID // 16) * (_GRID // 16) * 32
    return pl.pallas_call(
        _tail_body,
        out_shape=[jax.ShapeDtypeStruct((1, npo), _f32),
                   jax.ShapeDtypeStruct((1, 1), _f32)],
    )(parts, meta, cur, wm, bm, wq, bq, wk, bk, wv, bv,
      w1, b1, w2, b2, w3, b3, wp, bp)


def _dc_body4(x_ref, w_ref, b_ref, o_ref):
    x = x_ref[...]
    for p in range(4):
        o_ref[p] = jnp.maximum(
            jnp.dot(x, w_ref[p], precision=_HI) + b_ref[...], 0.0)


def _tc_deconv(xcat, wstack, bias, hw, co):
    return pl.pallas_call(
        _dc_body4,
        out_shape=jax.ShapeDtypeStruct((4, hw, co), _f32),
    )(xcat, wstack, bias)


def _conv_body(pl_ref, w_ref, b_ref, o_ref):
    acc = pl_ref[0] * w_ref[0:1, :]
    for t in range(1, 18):
        acc = acc + pl_ref[t] * w_ref[t:t + 1, :]
    o_ref[...] = acc + b_ref[...]


def _tc_conv_final(planes, wvec, bias):
    return pl.pallas_call(
        _conv_body,
        out_shape=jax.ShapeDtypeStruct((_GRID, _GRID), _f32),
    )(planes, wvec, bias)


def _shifts2(x):
    h, w, ci = x.shape
    xp = jnp.pad(x, ((1, 0), (1, 0), (0, 0)))
    mats = [xp[:h, :w], xp[:h, 1:], xp[1:, :w], x]
    return jnp.concatenate([m.reshape(h * w, ci) for m in mats], axis=1)


def _wstack2(kt):
    z = jnp.zeros_like(kt[0, 0])
    w00 = jnp.concatenate([kt[0, 0], kt[0, 2], kt[2, 0], kt[2, 2]], axis=0)
    w01 = jnp.concatenate([z, kt[0, 1], z, kt[2, 1]], axis=0)
    w10 = jnp.concatenate([z, z, kt[1, 0], kt[1, 2]], axis=0)
    w11 = jnp.concatenate([z, z, z, kt[1, 1]], axis=0)
    return jnp.stack([w00, w01, w10, w11])


def _interleave(y4, h, w, co):
    y = y4.reshape(2, 2, h, w, co)
    return y.transpose(2, 0, 3, 1, 4).reshape(2 * h, 2 * w, co)


def _shift_planes(x):
    h, w, ci = x.shape
    xp = jnp.pad(x, ((1, 1), (1, 1), (0, 0)))
    return jnp.stack([xp[di:di + h, dj:dj + w, c]
                      for di in range(3) for dj in range(3)
                      for c in range(ci)])


_P16 = np.zeros((16, 16), np.float32)
for _k_ in range(16):
    _P16[8 + (_k_ % 8), _k_] = 1.0
_PMAT = np.kron(np.eye(8, dtype=np.float32), _P16)


def _blockdiag(w16):
    return jnp.kron(jnp.eye(8, dtype=_f32), w16)


def kernel(node_features, netlist_metadata, sparse_adj_weight, params,
           sparse_adj_i, sparse_adj_j, current_node):
    p = params
    nf = node_features[0]
    wvec = sparse_adj_weight[0].reshape(_E)
    ii = sparse_adj_i[0].astype(jnp.int32)
    jj = sparse_adj_j[0].astype(jnp.int32)
    cur = current_node.astype(jnp.int32).reshape(1, 1)

    abs_, bas_, cbs_ = [], [], []
    for l in range(3):
        we = p['We'][l]
        be = p['be'][l]
        a_, b_, c_ = we[:_H], we[_H:2 * _H], we[2 * _H]
        abs_.append(jnp.concatenate([a_, b_], axis=1))
        bas_.append(jnp.concatenate([b_, a_], axis=1))
        cbs_.append(jnp.stack([jnp.concatenate([c_, c_]),
                               jnp.concatenate([be, be])]))
    zeros_n16 = jnp.zeros((_N, 16), _f32)

    pq, qp = _tc_k0(nf, p['Wf'], p['bf'].reshape(1, _H), abs_[0], bas_[0])
    parts = None
    for l in range(3):
        parts = _sc_layer(pq, qp, ii, jj, wvec, cbs_[l], zeros_n16)
        if l < 2:
            parts8 = parts.reshape(2 * _N * 16 // 128, 128)
            wab = _blockdiag(jnp.concatenate(
                [abs_[l + 1], jnp.zeros((_H, 16), _f32)], axis=0))
            wba = _blockdiag(jnp.concatenate(
                [bas_[l + 1], jnp.zeros((_H, 16), _f32)], axis=0))
            pq8, qp8 = _tc_prep(parts8, jnp.asarray(_PMAT), wab, wba)
            pq = pq8.reshape(_N, 16)
            qp = qp8.reshape(_N, 16)

    pol, val = _tc_tail(
        parts.reshape(2, _N, 16), netlist_metadata, cur,
        p['Wm'], p['bm'].reshape(1, -1),
        p['Wq'], p['bq'].reshape(1, -1),
        p['Wk'], p['bk'].reshape(1, -1),
        p['Wv'], p['bv'].reshape(1, -1),
        p['Wv1'], p['bv1'].reshape(1, -1),
        p['Wv2'], p['bv2'].reshape(1, -1),
        p['Wv3'], p['bv3'].reshape(1, -1),
        p['Wp'], p['bp'].reshape(1, -1))

    x = pol.reshape(_GRID // 16, _GRID // 16, 32)
    for kt, cb in ((p['K1'], p['c1']), (p['K2'], p['c2']),
                   (p['K3'], p['c3']), (p['K4'], p['c4'])):
        h, w, ci = x.shape
        co = kt.shape[3]
        y4 = _tc_deconv(_shifts2(x), _wstack2(kt), cb.reshape(1, -1),
                        hw=h * w, co=co)
        x = _interleave(y4, h, w, co)

    y = _tc_conv_final(_shift_planes(x), p['K5'].reshape(18, 1),
                       p['c5'].reshape(1, 1))
    logits = y.reshape(1, _GRID * _GRID)
    return logits, val


# speedup vs baseline: 17.4313x; 1.3767x over previous
import functools

import jax
import jax.numpy as jnp
import numpy as np
from jax import lax
from jax.experimental import pallas as pl
from jax.experimental.pallas import tpu as pltpu
from jax.experimental.pallas import tpu_sc as plsc

_N = 10000
_E = 320000
_DF = 128
_H = 8
_GRID = 128
_EPS = 1e-6
_f32 = jnp.float32
_HI = lax.Precision.HIGHEST

_NC = 2
_NS = 16
_NW = _NC * _NS
_K = 400
_SUB = 80
_NSUB = _K // _SUB
_EPW = _E // _NW
_NCH = _EPW // _K
_RPT = _N // _NS

_GDN = lax.GatherDimensionNumbers(
    offset_dims=(), collapsed_slice_dims=(0,), start_index_map=(0,))


def _g1d(x, idx):
    return lax.gather(x, idx[:, None], _GDN, slice_sizes=(1,),
                      mode=lax.GatherScatterMode.PROMISE_IN_BOUNDS)


def _sc_layer(pq, qp, ii, jj, wv, cb, zeros_n16):
    mesh = plsc.VectorSubcoreMesh(core_axis_name="c", subcore_axis_name="s")

    @functools.partial(
        pl.kernel,
        out_type=jax.ShapeDtypeStruct((2 * _N, 16), _f32),
        mesh=mesh,
        compiler_params=pltpu.CompilerParams(use_tc_tiling_on_sc=False),
        scratch_types=[
            pltpu.VMEM((_K,), jnp.int32),
            pltpu.VMEM((_K,), jnp.int32),
            pltpu.VMEM((_K,), _f32),
            pltpu.VMEM((_NSUB, _SUB), jnp.int32),
            pltpu.VMEM((_NSUB, _SUB), jnp.int32),
            pltpu.VMEM((_K, 16), _f32),
            pltpu.VMEM((_K, 16), _f32),
            pltpu.VMEM((_K, 16), _f32),
            pltpu.VMEM((2, 16), _f32),
            pltpu.VMEM_SHARED((_N, 16), _f32),
            pltpu.SemaphoreType.DMA,
            pltpu.SemaphoreType.DMA,
            pltpu.SemaphoreType.DMA,
        ],
    )
    def k(pq_h, qp_h, ii_h, jj_h, w_h, cb_h, z_h, out_h,
          i1, i2, wvv, si1, si2, b1, b2, hb, cbv, acc, s1, s2, s3):
        cid = lax.axis_index("c")
        sid = lax.axis_index("s")
        wid = cid * _NS + sid
        pltpu.sync_copy(cb_h, cbv)
        pltpu.sync_copy(z_h.at[pl.ds(sid * _RPT, _RPT)],
                        acc.at[pl.ds(sid * _RPT, _RPT)])
        plsc.subcore_barrier()
        base = wid * _EPW
        swap = lax.iota(jnp.int32, 16) ^ 8
        hi = lax.iota(jnp.int32, 16) >= 8
        cb0 = cbv[0, :]
        cb1 = cbv[1, :]

        def process(t):
            off = base + t * _K
            l1 = pltpu.async_copy(ii_h.at[pl.ds(off, _K)], i1, s1)
            l2 = pltpu.async_copy(jj_h.at[pl.ds(off, _K)], i2, s1)
            l3 = pltpu.async_copy(w_h.at[pl.ds(off, _K)], wvv, s1)
            l1.wait()
            l2.wait()
            l3.wait()
            gs = []
            for u in range(_NSUB):
                sl = pl.ds(u * _SUB, _SUB)
                gs.append(pltpu.async_copy(
                    pq_h.at[i1.at[sl]], b1.at[sl], s2))
                gs.append(pltpu.async_copy(
                    qp_h.at[i2.at[sl]], b2.at[sl], s2))
            for u in range(_NSUB):
                for m in range(_SUB // 16):
                    sl16 = pl.ds(u * _SUB + m * 16, 16)
                    si1[u, pl.ds(m * 16, 16)] = i1[sl16]
                    si2[u, pl.ds(m * 16, 16)] = i2[sl16]
            for g in gs:
                g.wait()
            for g in range(_K // 16):
                w16 = wvv[pl.ds(g * 16, 16)]
                for r16 in range(16):
                    r = g * 16 + r16
                    s = w16[r16]
                    mf = jnp.where(s != 0.0, 1.0, 0.0)
                    x = b1[r, :] + b2[r, :]
                    z = x * mf + (cb0 * s + cb1)
                    f = jnp.maximum(z, 0.0)
                    h2 = f + _g1d(f, swap)
                    hb[r, :] = jnp.where(hi, 1.0, h2)
            for u in range(_NSUB):
                sl = pl.ds(u * _SUB, _SUB)
                pltpu.async_copy(hb.at[sl], acc.at[si1.at[u]], s3, add=True)
                pltpu.async_copy(hb.at[sl], acc.at[si2.at[u]], s3, add=True)

        def drain_scatter():
            for u in range(_NSUB):
                sl = pl.ds(u * _SUB, _SUB)
                pltpu.make_async_copy(hb.at[sl], acc.at[si1.at[u]], s3).wait()
                pltpu.make_async_copy(hb.at[sl], acc.at[si2.at[u]], s3).wait()

        process(0)

        def body(t, carry):
            drain_scatter()
            process(t)
            return carry

        lax.fori_loop(1, _NCH, body, 0)
        drain_scatter()
        plsc.subcore_barrier()
        pltpu.sync_copy(acc.at[pl.ds(sid * _RPT, _RPT)],
                        out_h.at[pl.ds(cid * _N + sid * _RPT, _RPT)])

    return k(pq, qp, ii, jj, wv, cb, zeros_n16)


def _k0_body(nf_ref, wf_ref, bf_ref, ab_ref, ba_ref, pq_ref, qp_ref):
    h = jnp.maximum(
        jnp.dot(nf_ref[...], wf_ref[...], precision=_HI) + bf_ref[...], 0.0)
    pq_ref[...] = jnp.dot(h, ab_ref[...], precision=_HI)
    qp_ref[...] = jnp.dot(h, ba_ref[...], precision=_HI)


def _tc_k0(nf, wf, bf, ab, ba):
    blk = 2000
    return pl.pallas_call(
        _k0_body,
        grid=(_N // blk,),
        in_specs=[
            pl.BlockSpec((blk, _DF), lambda i: (i, 0)),
            pl.BlockSpec((_DF, _H), lambda i: (0, 0)),
            pl.BlockSpec((1, _H), lambda i: (0, 0)),
            pl.BlockSpec((_H, 16), lambda i: (0, 0)),
            pl.BlockSpec((_H, 16), lambda i: (0, 0)),
        ],
        out_specs=[
            pl.BlockSpec((blk, 16), lambda i: (i, 0)),
            pl.BlockSpec((blk, 16), lambda i: (i, 0)),
        ],
        out_shape=[jax.ShapeDtypeStruct((_N, 16), _f32)] * 2,
    )(nf, wf, bf, ab, ba)


def _prep_body(parts_ref, p_ref, wab_ref, wba_ref, pq_ref, qp_ref):
    half = _N * 16 // 128
    tsum = parts_ref[0:half, :] + parts_ref[half:2 * half, :]
    cnt = jnp.dot(tsum, p_ref[...], precision=_HI)
    h8 = tsum / (cnt + _EPS)
    pq_ref[...] = jnp.dot(h8, wab_ref[...], precision=_HI)
    qp_ref[...] = jnp.dot(h8, wba_ref[...], precision=_HI)


def _tc_prep(parts8, pmat, wab, wba):
    half = _N * 16 // 128
    return pl.pallas_call(
        _prep_body,
        out_shape=[jax.ShapeDtypeStruct((half, 128), _f32)] * 2,
    )(parts8, pmat, wab, wba)


def _bf(x):
    return x.astype(jnp.bfloat16).astype(_f32)


def _tail_body(parts_ref, meta_ref, cur_ref, wm, bm, wq, bq, wk, bk, wv, bv,
               w1, b1, w2, b2, w3, b3, wp, bp, pol_ref, val_ref):
    t = parts_ref[0] + parts_ref[1]
    h3 = t[:, :_H] / (t[:, _H:] + _EPS)
    ids = lax.broadcasted_iota(jnp.int32, (1, _N), 1)
    onehot = (ids == cur_ref[...]).astype(_f32)
    h_cur = jnp.dot(onehot, h3, precision=_HI)
    h_meta = jnp.maximum(
        jnp.dot(_bf(meta_ref[...]), _bf(wm[...]), precision=_HI)
        + bm[...], 0.0)
    q = jnp.dot(_bf(h_cur), _bf(wq[...]), precision=_HI) + bq[...]
    kk = jnp.dot(h3, wk[...], precision=_HI) + bk[...]
    vv = jnp.dot(h3, wv[...], precision=_HI) + bv[...]
    s = jnp.sum(_bf(kk) * _bf(q), axis=1, keepdims=True)
    m = jnp.max(s, axis=0, keepdims=True)
    e = jnp.exp(s - m)
    att = e / jnp.sum(e, axis=0, keepdims=True)
    h_att = jnp.sum(_bf(vv) * _bf(att), axis=0, keepdims=True)
    h = jnp.concatenate([h_meta, h_cur, h_att], axis=1)
    u = jnp.maximum(
        jnp.dot(_bf(h), _bf(w1[...]), precision=_HI) + b1[...], 0.0)
    u = jnp.maximum(
        jnp.dot(_bf(u), _bf(w2[...]), precision=_HI) + b2[...], 0.0)
    val_ref[...] = jnp.dot(_bf(u), _bf(w3[...]), precision=_HI) + b3[...]
    pol_ref[...] = jnp.maximum(
        jnp.dot(_bf(h), _bf(wp[...]), precision=_HI) + bp[...], 0.0)


def _tc_tail(parts, meta, cur, wm, bm, wq, bq, wk, bk, wv, bv,
             w1, b1, w2, b2, w3, b3, wp, bp):
    npo = (_GRID // 16) * (_GRID // 16) * 32
    return pl.pallas_call(
        _tail_body,
        out_shape=[jax.ShapeDtypeStruct((1, npo), _f32),
                   jax.ShapeDtypeStruct((1, 1), _f32)],
    )(parts, meta, cur, wm, bm, wq, bq, wk, bk, wv, bv,
      w1, b1, w2, b2, w3, b3, wp, bp)


def _dc_body4(x_ref, w_ref, b_ref, o_ref):
    x = x_ref[...]
    for p in range(4):
        o_ref[p] = jnp.maximum(
            jnp.dot(x, w_ref[p], precision=_HI) + b_ref[...], 0.0)


def _tc_deconv(xcat, wstack, bias, hw, co):
    return pl.pallas_call(
        _dc_body4,
        out_shape=jax.ShapeDtypeStruct((4, hw, co), _f32),
    )(xcat, wstack, bias)


def _conv_body(pl_ref, w_ref, b_ref, o_ref):
    acc = pl_ref[0] * w_ref[0:1, :]
    for t in range(1, 18):
        acc = acc + pl_ref[t] * w_ref[t:t + 1, :]
    o_ref[...] = acc + b_ref[...]


def _tc_conv_final(planes, wvec, bias):
    return pl.pallas_call(
        _conv_body,
        out_shape=jax.ShapeDtypeStruct((_GRID, _GRID), _f32),
    )(planes, wvec, bias)


def _shifts2(x):
    h, w, ci = x.shape
    xp = jnp.pad(x, ((1, 0), (1, 0), (0, 0)))
    mats = [xp[:h, :w], xp[:h, 1:], xp[1:, :w], x]
    return jnp.concatenate([m.reshape(h * w, ci) for m in mats], axis=1)


def _wstack2(kt):
    z = jnp.zeros_like(kt[0, 0])
    w00 = jnp.concatenate([kt[0, 0], kt[0, 2], kt[2, 0], kt[2, 2]], axis=0)
    w01 = jnp.concatenate([z, kt[0, 1], z, kt[2, 1]], axis=0)
    w10 = jnp.concatenate([z, z, kt[1, 0], kt[1, 2]], axis=0)
    w11 = jnp.concatenate([z, z, z, kt[1, 1]], axis=0)
    return jnp.stack([w00, w01, w10, w11])


def _interleave(y4, h, w, co):
    y = y4.reshape(2, 2, h, w, co)
    return y.transpose(2, 0, 3, 1, 4).reshape(2 * h, 2 * w, co)


def _shift_planes(x):
    h, w, ci = x.shape
    xp = jnp.pad(x, ((1, 1), (1, 1), (0, 0)))
    return jnp.stack([xp[di:di + h, dj:dj + w, c]
                      for di in range(3) for dj in range(3)
                      for c in range(ci)])


_P16 = np.zeros((16, 16), np.float32)
for _k_ in range(16):
    _P16[8 + (_k_ % 8), _k_] = 1.0
_PMAT = np.kron(np.eye(8, dtype=np.float32), _P16)


def _blockdiag(w16):
    return jnp.kron(jnp.eye(8, dtype=_f32), w16)


def kernel(node_features, netlist_metadata, sparse_adj_weight, params,
           sparse_adj_i, sparse_adj_j, current_node):
    p = params
    nf = node_features[0]
    wvec = sparse_adj_weight[0].reshape(_E)
    ii = sparse_adj_i[0].astype(jnp.int32)
    jj = sparse_adj_j[0].astype(jnp.int32)
    cur = current_node.astype(jnp.int32).reshape(1, 1)

    abs_, bas_, cbs_ = [], [], []
    for l in range(3):
        we = p['We'][l]
        be = p['be'][l]
        a_, b_, c_ = 0.5 * we[:_H], 0.5 * we[_H:2 * _H], 0.5 * we[2 * _H]
        abs_.append(jnp.concatenate([a_, b_], axis=1))
        bas_.append(jnp.concatenate([b_, a_], axis=1))
        cbs_.append(jnp.stack([jnp.concatenate([c_, c_]),
                               0.5 * jnp.concatenate([be, be])]))
    zeros_n16 = jnp.zeros((_N, 16), _f32)

    pq, qp = _tc_k0(nf, p['Wf'], p['bf'].reshape(1, _H), abs_[0], bas_[0])
    parts = None
    for l in range(3):
        parts = _sc_layer(pq, qp, ii, jj, wvec, cbs_[l], zeros_n16)
        if l < 2:
            parts8 = parts.reshape(2 * _N * 16 // 128, 128)
            wab = _blockdiag(jnp.concatenate(
                [abs_[l + 1], jnp.zeros((_H, 16), _f32)], axis=0))
            wba = _blockdiag(jnp.concatenate(
                [bas_[l + 1], jnp.zeros((_H, 16), _f32)], axis=0))
            pq8, qp8 = _tc_prep(parts8, jnp.asarray(_PMAT), wab, wba)
            pq = pq8.reshape(_N, 16)
            qp = qp8.reshape(_N, 16)

    pol, val = _tc_tail(
        parts.reshape(2, _N, 16), netlist_metadata, cur,
        p['Wm'], p['bm'].reshape(1, -1),
        p['Wq'], p['bq'].reshape(1, -1),
        p['Wk'], p['bk'].reshape(1, -1),
        p['Wv'], p['bv'].reshape(1, -1),
        p['Wv1'], p['bv1'].reshape(1, -1),
        p['Wv2'], p['bv2'].reshape(1, -1),
        p['Wv3'], p['bv3'].reshape(1, -1),
        p['Wp'], p['bp'].reshape(1, -1))

    x = pol.reshape(_GRID // 16, _GRID // 16, 32)
    for kt, cb in ((p['K1'], p['c1']), (p['K2'], p['c2']),
                   (p['K3'], p['c3']), (p['K4'], p['c4'])):
        h, w, ci = x.shape
        co = kt.shape[3]
        y4 = _tc_deconv(_shifts2(x), _wstack2(kt), cb.reshape(1, -1),
                        hw=h * w, co=co)
        x = _interleave(y4, h, w, co)

    y = _tc_conv_final(_shift_planes(x), p['K5'].reshape(18, 1),
                       p['c5'].reshape(1, 1))
    logits = y.reshape(1, _GRID * _GRID)
    return logits, val


# speedup vs baseline: 19.5659x; 1.1225x over previous
import functools

import jax
import jax.numpy as jnp
import numpy as np
from jax import lax
from jax.experimental import pallas as pl
from jax.experimental.pallas import tpu as pltpu
from jax.experimental.pallas import tpu_sc as plsc

_N = 10000
_E = 320000
_DF = 128
_H = 8
_GRID = 128
_EPS = 1e-6
_f32 = jnp.float32
_HI = lax.Precision.HIGHEST

_NC = 2
_NS = 16
_NW = _NC * _NS
_K = 80
_EPW = _E // _NW
_NCH = _EPW // _K
_RPT = _N // _NS

_GDN = lax.GatherDimensionNumbers(
    offset_dims=(), collapsed_slice_dims=(0,), start_index_map=(0,))


def _g1d(x, idx):
    return lax.gather(x, idx[:, None], _GDN, slice_sizes=(1,),
                      mode=lax.GatherScatterMode.PROMISE_IN_BOUNDS)


def _sc_layer(pq, qp, ii, jj, wv, cb, zeros_n16):
    mesh = plsc.VectorSubcoreMesh(core_axis_name="c", subcore_axis_name="s")

    @functools.partial(
        pl.kernel,
        out_type=jax.ShapeDtypeStruct((2 * _N, 16), _f32),
        mesh=mesh,
        compiler_params=pltpu.CompilerParams(use_tc_tiling_on_sc=False),
        scratch_types=[
            pltpu.VMEM((4, _K), jnp.int32),
            pltpu.VMEM((4, _K), jnp.int32),
            pltpu.VMEM((4, _K), _f32),
            pltpu.VMEM((2, _K), jnp.int32),
            pltpu.VMEM((2, _K), jnp.int32),
            pltpu.VMEM((2, _K, 16), _f32),
            pltpu.VMEM((2, _K, 16), _f32),
            pltpu.VMEM((2, _K, 16), _f32),
            pltpu.VMEM((2, 16), _f32),
            pltpu.VMEM_SHARED((_N, 16), _f32),
            pltpu.SemaphoreType.DMA,
            pltpu.SemaphoreType.DMA,
            pltpu.SemaphoreType.DMA,
        ],
    )
    def k(pq_h, qp_h, ii_h, jj_h, w_h, cb_h, z_h, out_h,
          i1r, i2r, wvr, si1, si2, b1r, b2r, hbr, cbv, acc, s1, s2, s3):
        cid = lax.axis_index("c")
        sid = lax.axis_index("s")
        wid = cid * _NS + sid
        pltpu.sync_copy(cb_h, cbv)
        pltpu.sync_copy(z_h.at[pl.ds(sid * _RPT, _RPT)],
                        acc.at[pl.ds(sid * _RPT, _RPT)])
        plsc.subcore_barrier()
        base = wid * _EPW
        swap = lax.iota(jnp.int32, 16) ^ 8
        hi = lax.iota(jnp.int32, 16) >= 8
        cb0 = cbv[0, :]
        cb1 = cbv[1, :]

        def issue_load(t, q):
            off = base + t * _K
            pltpu.async_copy(ii_h.at[pl.ds(off, _K)], i1r.at[q], s1)
            pltpu.async_copy(jj_h.at[pl.ds(off, _K)], i2r.at[q], s1)
            pltpu.async_copy(w_h.at[pl.ds(off, _K)], wvr.at[q], s1)

        def wait_load(t, q):
            off = base + t * _K
            pltpu.make_async_copy(ii_h.at[pl.ds(off, _K)], i1r.at[q], s1).wait()
            pltpu.make_async_copy(jj_h.at[pl.ds(off, _K)], i2r.at[q], s1).wait()
            pltpu.make_async_copy(w_h.at[pl.ds(off, _K)], wvr.at[q], s1).wait()

        def issue_gather(p, q):
            pltpu.async_copy(pq_h.at[i1r.at[q]], b1r.at[p], s2)
            pltpu.async_copy(qp_h.at[i2r.at[q]], b2r.at[p], s2)

        def wait_gather(p, q):
            pltpu.make_async_copy(pq_h.at[i1r.at[q]], b1r.at[p], s2).wait()
            pltpu.make_async_copy(qp_h.at[i2r.at[q]], b2r.at[p], s2).wait()

        def issue_scatter(p):
            pltpu.async_copy(hbr.at[p], acc.at[si1.at[p]], s3, add=True)
            pltpu.async_copy(hbr.at[p], acc.at[si2.at[p]], s3, add=True)

        def drain_scatter(p):
            pltpu.make_async_copy(hbr.at[p], acc.at[si1.at[p]], s3).wait()
            pltpu.make_async_copy(hbr.at[p], acc.at[si2.at[p]], s3).wait()

        def compute(p, q):
            for m in range(_K // 16):
                sl16 = pl.ds(m * 16, 16)
                si1[p, sl16] = i1r[q, sl16]
                si2[p, sl16] = i2r[q, sl16]
            for g in range(_K // 16):
                w16 = wvr[q, pl.ds(g * 16, 16)]
                for r16 in range(16):
                    r = g * 16 + r16
                    s = w16[r16]
                    mf = jnp.where(s != 0.0, 1.0, 0.0)
                    x = b1r[p, r, :] + b2r[p, r, :]
                    z = x * mf + (cb0 * s + cb1)
                    f = jnp.maximum(z, 0.0)
                    h2 = f + _g1d(f, swap)
                    hbr[p, r, :] = jnp.where(hi, 1.0, h2)

        def step(t, q4, drain):
            p = q4 % 2
            pn = 1 - p
            qn = (q4 + 1) % 4
            qL = (q4 + 3) % 4
            tn = jnp.minimum(t + 1, _NCH - 1)
            tL = jnp.minimum(t + 3, _NCH - 1)
            wait_load(tn, qn)
            issue_gather(pn, qn)
            wait_gather(p, q4)
            if drain:
                drain_scatter(p)
            compute(p, q4)
            issue_scatter(p)
            issue_load(tL, qL)

        issue_load(0, 0)
        issue_load(1, 1)
        issue_load(2, 2)
        wait_load(0, 0)
        issue_gather(0, 0)
        step(0, 0, drain=False)
        step(1, 1, drain=False)
        step(2, 2, drain=True)
        step(3, 3, drain=True)
        step(4, 0, drain=True)

        def body(u, carry):
            c = 5 + 4 * u
            step(c, 1, drain=True)
            step(c + 1, 2, drain=True)
            step(c + 2, 3, drain=True)
            step(c + 3, 0, drain=True)
            return carry

        lax.fori_loop(0, (_NCH - 5) // 4, body, 0)
        drain_scatter(1)
        drain_scatter(0)
        wait_load(_NCH - 1, 2)
        wait_load(_NCH - 1, 3)
        wait_gather(1, 1)
        plsc.subcore_barrier()
        pltpu.sync_copy(acc.at[pl.ds(sid * _RPT, _RPT)],
                        out_h.at[pl.ds(cid * _N + sid * _RPT, _RPT)])

    return k(pq, qp, ii, jj, wv, cb, zeros_n16)


def _k0_body(nf_ref, wf_ref, bf_ref, ab_ref, ba_ref, pq_ref, qp_ref):
    h = jnp.maximum(
        jnp.dot(nf_ref[...], wf_ref[...], precision=_HI) + bf_ref[...], 0.0)
    pq_ref[...] = jnp.dot(h, ab_ref[...], precision=_HI)
    qp_ref[...] = jnp.dot(h, ba_ref[...], precision=_HI)


def _tc_k0(nf, wf, bf, ab, ba):
    blk = 2000
    return pl.pallas_call(
        _k0_body,
        grid=(_N // blk,),
        in_specs=[
            pl.BlockSpec((blk, _DF), lambda i: (i, 0)),
            pl.BlockSpec((_DF, _H), lambda i: (0, 0)),
            pl.BlockSpec((1, _H), lambda i: (0, 0)),
            pl.BlockSpec((_H, 16), lambda i: (0, 0)),
            pl.BlockSpec((_H, 16), lambda i: (0, 0)),
        ],
        out_specs=[
            pl.BlockSpec((blk, 16), lambda i: (i, 0)),
            pl.BlockSpec((blk, 16), lambda i: (i, 0)),
        ],
        out_shape=[jax.ShapeDtypeStruct((_N, 16), _f32)] * 2,
    )(nf, wf, bf, ab, ba)


def _prep_body(parts_ref, p_ref, wab_ref, wba_ref, pq_ref, qp_ref):
    half = _N * 16 // 128
    tsum = parts_ref[0:half, :] + parts_ref[half:2 * half, :]
    cnt = jnp.dot(tsum, p_ref[...], precision=_HI)
    h8 = tsum / (cnt + _EPS)
    pq_ref[...] = jnp.dot(h8, wab_ref[...], precision=_HI)
    qp_ref[...] = jnp.dot(h8, wba_ref[...], precision=_HI)


def _tc_prep(parts8, pmat, wab, wba):
    half = _N * 16 // 128
    return pl.pallas_call(
        _prep_body,
        out_shape=[jax.ShapeDtypeStruct((half, 128), _f32)] * 2,
    )(parts8, pmat, wab, wba)


def _bf(x):
    return x.astype(jnp.bfloat16).astype(_f32)


def _tail_body(parts_ref, meta_ref, cur_ref, wm, bm, wq, bq, wk, bk, wv, bv,
               w1, b1, w2, b2, w3, b3, wp, bp, pol_ref, val_ref):
    t = parts_ref[0] + parts_ref[1]
    h3 = t[:, :_H] / (t[:, _H:] + _EPS)
    ids = lax.broadcasted_iota(jnp.int32, (1, _N), 1)
    onehot = (ids == cur_ref[...]).astype(_f32)
    h_cur = jnp.dot(onehot, h3, precision=_HI)
    h_meta = jnp.maximum(
        jnp.dot(_bf(meta_ref[...]), _bf(wm[...]), precision=_HI)
        + bm[...], 0.0)
    q = jnp.dot(_bf(h_cur), _bf(wq[...]), precision=_HI) + bq[...]
    kk = jnp.dot(h3, wk[...], precision=_HI) + bk[...]
    vv = jnp.dot(h3, wv[...], precision=_HI) + bv[...]
    s = jnp.sum(_bf(kk) * _bf(q), axis=1, keepdims=True)
    m = jnp.max(s, axis=0, keepdims=True)
    e = jnp.exp(s - m)
    att = e / jnp.sum(e, axis=0, keepdims=True)
    h_att = jnp.sum(_bf(vv) * _bf(att), axis=0, keepdims=True)
    h = jnp.concatenate([h_meta, h_cur, h_att], axis=1)
    u = jnp.maximum(
        jnp.dot(_bf(h), _bf(w1[...]), precision=_HI) + b1[...], 0.0)
    u = jnp.maximum(
        jnp.dot(_bf(u), _bf(w2[...]), precision=_HI) + b2[...], 0.0)
    val_ref[...] = jnp.dot(_bf(u), _bf(w3[...]), precision=_HI) + b3[...]
    pol_ref[...] = jnp.maximum(
        jnp.dot(_bf(h), _bf(wp[...]), precision=_HI) + bp[...], 0.0)


def _tc_tail(parts, meta, cur, wm, bm, wq, bq, wk, bk, wv, bv,
             w1, b1, w2, b2, w3, b3, wp, bp):
    npo = (_GRID // 16) * (_GRID // 16) * 32
    return pl.pallas_call(
        _tail_body,
        out_shape=[jax.ShapeDtypeStruct((1, npo), _f32),
                   jax.ShapeDtypeStruct((1, 1), _f32)],
    )(parts, meta, cur, wm, bm, wq, bq, wk, bk, wv, bv,
      w1, b1, w2, b2, w3, b3, wp, bp)


def _dc_body4(x_ref, w_ref, b_ref, o_ref):
    x = x_ref[...]
    for p in range(4):
        o_ref[p] = jnp.maximum(
            jnp.dot(x, w_ref[p], precision=_HI) + b_ref[...], 0.0)


def _tc_deconv(xcat, wstack, bias, hw, co):
    return pl.pallas_call(
        _dc_body4,
        out_shape=jax.ShapeDtypeStruct((4, hw, co), _f32),
    )(xcat, wstack, bias)


def _conv_body(pl_ref, w_ref, b_ref, o_ref):
    acc = pl_ref[0] * w_ref[0:1, :]
    for t in range(1, 18):
        acc = acc + pl_ref[t] * w_ref[t:t + 1, :]
    o_ref[...] = acc + b_ref[...]


def _tc_conv_final(planes, wvec, bias):
    return pl.pallas_call(
        _conv_body,
        out_shape=jax.ShapeDtypeStruct((_GRID, _GRID), _f32),
    )(planes, wvec, bias)


def _shifts2(x):
    h, w, ci = x.shape
    xp = jnp.pad(x, ((1, 0), (1, 0), (0, 0)))
    mats = [xp[:h, :w], xp[:h, 1:], xp[1:, :w], x]
    return jnp.concatenate([m.reshape(h * w, ci) for m in mats], axis=1)


def _wstack2(kt):
    z = jnp.zeros_like(kt[0, 0])
    w00 = jnp.concatenate([kt[0, 0], kt[0, 2], kt[2, 0], kt[2, 2]], axis=0)
    w01 = jnp.concatenate([z, kt[0, 1], z, kt[2, 1]], axis=0)
    w10 = jnp.concatenate([z, z, kt[1, 0], kt[1, 2]], axis=0)
    w11 = jnp.concatenate([z, z, z, kt[1, 1]], axis=0)
    return jnp.stack([w00, w01, w10, w11])


def _interleave(y4, h, w, co):
    y = y4.reshape(2, 2, h, w, co)
    return y.transpose(2, 0, 3, 1, 4).reshape(2 * h, 2 * w, co)


def _shift_planes(x):
    h, w, ci = x.shape
    xp = jnp.pad(x, ((1, 1), (1, 1), (0, 0)))
    return jnp.stack([xp[di:di + h, dj:dj + w, c]
                      for di in range(3) for dj in range(3)
                      for c in range(ci)])


_P16 = np.zeros((16, 16), np.float32)
for _k_ in range(16):
    _P16[8 + (_k_ % 8), _k_] = 1.0
_PMAT = np.kron(np.eye(8, dtype=np.float32), _P16)


def _blockdiag(w16):
    return jnp.kron(jnp.eye(8, dtype=_f32), w16)


def kernel(node_features, netlist_metadata, sparse_adj_weight, params,
           sparse_adj_i, sparse_adj_j, current_node):
    p = params
    nf = node_features[0]
    wvec = sparse_adj_weight[0].reshape(_E)
    ii = sparse_adj_i[0].astype(jnp.int32)
    jj = sparse_adj_j[0].astype(jnp.int32)
    cur = current_node.astype(jnp.int32).reshape(1, 1)

    abs_, bas_, cbs_ = [], [], []
    for l in range(3):
        we = p['We'][l]
        be = p['be'][l]
        a_, b_, c_ = 0.5 * we[:_H], 0.5 * we[_H:2 * _H], 0.5 * we[2 * _H]
        abs_.append(jnp.concatenate([a_, b_], axis=1))
        bas_.append(jnp.concatenate([b_, a_], axis=1))
        cbs_.append(jnp.stack([jnp.concatenate([c_, c_]),
                               0.5 * jnp.concatenate([be, be])]))
    zeros_n16 = jnp.zeros((_N, 16), _f32)

    pq, qp = _tc_k0(nf, p['Wf'], p['bf'].reshape(1, _H), abs_[0], bas_[0])
    parts = None
    for l in range(3):
        parts = _sc_layer(pq, qp, ii, jj, wvec, cbs_[l], zeros_n16)
        if l < 2:
            parts8 = parts.reshape(2 * _N * 16 // 128, 128)
            wab = _blockdiag(jnp.concatenate(
                [abs_[l + 1], jnp.zeros((_H, 16), _f32)], axis=0))
            wba = _blockdiag(jnp.concatenate(
                [bas_[l + 1], jnp.zeros((_H, 16), _f32)], axis=0))
            pq8, qp8 = _tc_prep(parts8, jnp.asarray(_PMAT), wab, wba)
            pq = pq8.reshape(_N, 16)
            qp = qp8.reshape(_N, 16)

    pol, val = _tc_tail(
        parts.reshape(2, _N, 16), netlist_metadata, cur,
        p['Wm'], p['bm'].reshape(1, -1),
        p['Wq'], p['bq'].reshape(1, -1),
        p['Wk'], p['bk'].reshape(1, -1),
        p['Wv'], p['bv'].reshape(1, -1),
        p['Wv1'], p['bv1'].reshape(1, -1),
        p['Wv2'], p['bv2'].reshape(1, -1),
        p['Wv3'], p['bv3'].reshape(1, -1),
        p['Wp'], p['bp'].reshape(1, -1))

    x = pol.reshape(_GRID // 16, _GRID // 16, 32)
    for kt, cb in ((p['K1'], p['c1']), (p['K2'], p['c2']),
                   (p['K3'], p['c3']), (p['K4'], p['c4'])):
        h, w, ci = x.shape
        co = kt.shape[3]
        y4 = _tc_deconv(_shifts2(x), _wstack2(kt), cb.reshape(1, -1),
                        hw=h * w, co=co)
        x = _interleave(y4, h, w, co)

    y = _tc_conv_final(_shift_planes(x), p['K5'].reshape(18, 1),
                       p['c5'].reshape(1, 1))
    logits = y.reshape(1, _GRID * _GRID)
    return logits, val
